# baseline (speedup 1.0000x reference)
import functools

import jax
import jax.numpy as jnp
from jax import lax
from jax.experimental import pallas as pl
from jax.experimental.pallas import tpu as pltpu
from jax.experimental.pallas import tpu_sc as plsc

N = 10000
D = 128
H = 16
C = 7
E = 320000

_NC, _NS = 2, 16
_NT = _NC * _NS
_CHUNK = 128
_CPT = 80
_EPAD = _NT * _CPT * _CHUNK
_NROW = _EPAD // _CHUNK
_ROWS = 10240
_RPT = _ROWS // _NS
_DEG = _ROWS
_NBUF = 8
_PRE = 4
_HCPT = _NROW // _NS

_F32 = jnp.float32


def _splat(ref, i):
    return plsc.load_gather(ref, [jnp.full((16,), i, jnp.int32)])


def _newton_rsqrt(d):
    i = plsc.bitcast(d, jnp.int32)
    i = 0x5F3759DF - lax.shift_right_arithmetic(i, 1)
    y = plsc.bitcast(i, _F32)
    for _ in range(3):
        y = y * (1.5 - 0.5 * d * y * y)
    return y


def _load_edge_idx(src2d, dst2d, srcv, dstv, t):
    pltpu.sync_copy(src2d.at[pl.ds(t * _CPT, _CPT)], srcv)
    pltpu.sync_copy(dst2d.at[pl.ds(t * _CPT, _CPT)], dstv)


def _zero_spacc_slice(rows, spacc, s):
    zeros16 = jnp.zeros((16,), _F32)

    def zero_row(i, carry):
        for k in range(8):
            rows[0, i * 8 + k] = zeros16
        return carry

    lax.fori_loop(0, _CHUNK // 8, zero_row, 0)
    for q in range(_RPT // _CHUNK):
        pltpu.sync_copy(rows.at[0],
                        spacc.at[pl.ds(s * _RPT + q * _CHUNK, _CHUNK)])


def _ring(gtab, spacc, srcv, dstv, rows, gsems, ssems):
    gcp = [None] * _NBUF
    scp = [None] * _NBUF
    for b in range(_PRE):
        gcp[b] = pltpu.async_copy(gtab.at[srcv.at[b]], rows.at[b], gsems[b])
    for j in range(_CPT):
        b = j % _NBUF
        gcp[b].wait()
        scp[b] = pltpu.async_copy(
            rows.at[b], spacc.at[dstv.at[j]], ssems[b], add=True)
        f = j + _PRE
        if f < _CPT:
            bf = f % _NBUF
            if f >= _NBUF:
                scp[bf].wait()
            gcp[bf] = pltpu.async_copy(gtab.at[srcv.at[f]], rows.at[bf],
                                       gsems[bf])
    for j in range(max(_CPT - _NBUF, 0), _CPT):
        scp[j % _NBUF].wait()


def _agg1_body(h1p, src2d, dst2d, aggp, dinv_out,
               srcv, dstv, dsthist, hist, rows, degv, dinvv, hbuf, ubuf,
               dsem, gtab, spacc, spdeg, *sems):
    gsems = sems[:_NBUF]
    ssems = sems[_NBUF:]
    c = lax.axis_index("c")
    s = lax.axis_index("s")
    t = c * _NS + s
    _load_edge_idx(src2d, dst2d, srcv, dstv, t)
    zeros16 = jnp.zeros((16,), _F32)

    def hzero(i, carry):
        for k in range(8):
            hist[pl.ds((i * 8 + k) * 16, 16)] = zeros16
        return carry

    lax.fori_loop(0, _DEG // 128, hzero, 0)
    ones16 = jnp.ones((16,), _F32)

    def hchunk(j, carry):
        for k in range(_CHUNK // 16):
            idx = dsthist[j, pl.ds(k * 16, 16)]
            plsc.addupdate_scatter(hist, [idx], ones16)
        return carry

    hq = _HCPT // 4
    for q in range(4):
        pltpu.sync_copy(dst2d.at[pl.ds(s * _HCPT + q * hq, hq)], dsthist)
        lax.fori_loop(0, hq, hchunk, 0)
    pltpu.sync_copy(hist, spdeg.at[s])
    pltpu.sync_copy(h1p.at[pl.ds(s * _RPT, _RPT)], hbuf)
    _zero_spacc_slice(rows, spacc, s)
    plsc.subcore_barrier()
    for half in range(2):
        dcp = []
        for k in range(8):
            dcp.append(pltpu.async_copy(
                spdeg.at[half * 8 + k].at[pl.ds(s * _RPT, _RPT)],
                degv.at[k], dsem))
        for cp in dcp:
            cp.wait()
        for v in range(_RPT // 16):
            sl = pl.ds(v * 16, 16)
            acc = jnp.ones((16,), _F32) if half == 0 else dinvv[sl]
            for k in range(8):
                acc = acc + degv[k, sl]
            dinvv[sl] = acc if half == 0 else _newton_rsqrt(acc)
    pltpu.sync_copy(dinvv, dinv_out.at[pl.ds(s * _RPT, _RPT)])

    def scale_row(i, carry):
        dv = _splat(dinvv, i)
        ubuf[i] = hbuf[i] * dv
        return carry

    lax.fori_loop(0, _RPT, scale_row, 0)
    pltpu.sync_copy(ubuf, gtab.at[pl.ds(s * _RPT, _RPT)])
    plsc.subcore_barrier()
    _ring(gtab, spacc, srcv, dstv, rows, gsems, ssems)
    plsc.subcore_barrier()
    pltpu.sync_copy(spacc.at[pl.ds(s * _RPT, _RPT)],
                    aggp.at[c].at[pl.ds(s * _RPT, _RPT)])


def _agg2_body(h1p, src2d, dst2d, dinv_hbm, agg1p, b1, aggp, w_out,
               srcv, dstv, rows, dinvv, hbuf, p0buf, p1buf, ubuf, wbuf, bbuf,
               gtab, spacc, *sems):
    gsems = sems[:_NBUF]
    ssems = sems[_NBUF:]
    c = lax.axis_index("c")
    s = lax.axis_index("s")
    t = c * _NS + s
    _load_edge_idx(src2d, dst2d, srcv, dstv, t)
    pltpu.sync_copy(dinv_hbm.at[pl.ds(s * _RPT, _RPT)], dinvv)
    pltpu.sync_copy(h1p.at[pl.ds(s * _RPT, _RPT)], hbuf)
    pltpu.sync_copy(agg1p.at[0].at[pl.ds(s * _RPT, _RPT)], p0buf)
    pltpu.sync_copy(agg1p.at[1].at[pl.ds(s * _RPT, _RPT)], p1buf)
    pltpu.sync_copy(b1, bbuf)
    _zero_spacc_slice(rows, spacc, s)
    b1v = bbuf[...]

    def urow(i, carry):
        dv = _splat(dinvv, i)
        g1r = hbuf[i] * dv
        srow = p0buf[i] + p1buf[i] + g1r
        z = jnp.maximum(srow * dv + b1v, 0.0)
        u = z * dv
        ubuf[i] = u
        wbuf[i] = u * dv
        return carry

    lax.fori_loop(0, _RPT, urow, 0)
    pltpu.sync_copy(ubuf, gtab.at[pl.ds(s * _RPT, _RPT)])
    pltpu.sync_copy(wbuf, w_out.at[pl.ds(s * _RPT, _RPT)])
    plsc.subcore_barrier()
    _ring(gtab, spacc, srcv, dstv, rows, gsems, ssems)
    plsc.subcore_barrier()
    pltpu.sync_copy(spacc.at[pl.ds(s * _RPT, _RPT)], ubuf)

    def srow_(i, carry):
        ubuf[i] = ubuf[i] * _splat(dinvv, i)
        return carry

    lax.fori_loop(0, _RPT, srow_, 0)
    pltpu.sync_copy(ubuf, aggp.at[c].at[pl.ds(s * _RPT, _RPT)])


def _prep_body(x_ref, w1_ref, h1_ref):
    h1 = jnp.dot(x_ref[...], w1_ref[...], preferred_element_type=_F32)
    h1_ref[:N, :] = h1
    h1_ref[N:, :] = jnp.zeros((_ROWS - N, H), _F32)


def _final_body(p0_ref, p1_ref, w_ref, w2_ref, b2_ref, out_ref):
    t = p0_ref[...] + p1_ref[...] + w_ref[...]
    a2 = (jnp.dot(t, w2_ref[...], preferred_element_type=_F32)
          + b2_ref[...][None, :])
    col = lax.broadcasted_iota(jnp.int32, (N, H), 1)
    valid = col < C
    am = jnp.where(valid, a2, -1e30)
    m = jnp.max(am, axis=1, keepdims=True)
    e = jnp.where(valid, jnp.exp(am - m), 0.0)
    ssum = jnp.sum(e, axis=1, keepdims=True)
    ls = a2 - m - jnp.log(ssum)
    out_ref[...] = ls[:, :C]


def _build(interpret: bool = False):
    mesh = plsc.VectorSubcoreMesh(
        core_axis_name="c", subcore_axis_name="s",
        num_cores=_NC, num_subcores=_NS)
    sc_params = pltpu.CompilerParams(
        use_tc_tiling_on_sc=False, needs_layout_passes=False)

    agg1_k = pl.kernel(
        _agg1_body,
        out_type=[jax.ShapeDtypeStruct((_NC, _ROWS, H), _F32),
                  jax.ShapeDtypeStruct((_ROWS,), _F32)],
        mesh=mesh,
        scratch_types=[
            pltpu.VMEM((_CPT, _CHUNK), jnp.int32),
            pltpu.VMEM((_CPT, _CHUNK), jnp.int32),
            pltpu.VMEM((_HCPT // 4, _CHUNK), jnp.int32),
            pltpu.VMEM((_DEG,), _F32),
            pltpu.VMEM((_NBUF, _CHUNK, H), _F32),
            pltpu.VMEM((8, _RPT), _F32),
            pltpu.VMEM((_RPT,), _F32),
            pltpu.VMEM((_RPT, H), _F32),
            pltpu.VMEM((_RPT, H), _F32),
            pltpu.SemaphoreType.DMA,
            pltpu.VMEM_SHARED((_ROWS, H), _F32),
            pltpu.VMEM_SHARED((_ROWS, H), _F32),
            pltpu.VMEM_SHARED((_NS, _DEG), _F32),
        ] + [pltpu.SemaphoreType.DMA] * (2 * _NBUF),
        compiler_params=sc_params,
        interpret=interpret,
    )

    agg2_k = pl.kernel(
        _agg2_body,
        out_type=[jax.ShapeDtypeStruct((_NC, _ROWS, H), _F32),
                  jax.ShapeDtypeStruct((_ROWS, H), _F32)],
        mesh=mesh,
        scratch_types=[
            pltpu.VMEM((_CPT, _CHUNK), jnp.int32),
            pltpu.VMEM((_CPT, _CHUNK), jnp.int32),
            pltpu.VMEM((_NBUF, _CHUNK, H), _F32),
            pltpu.VMEM((_RPT,), _F32),
            pltpu.VMEM((_RPT, H), _F32),
            pltpu.VMEM((_RPT, H), _F32),
            pltpu.VMEM((_RPT, H), _F32),
            pltpu.VMEM((_RPT, H), _F32),
            pltpu.VMEM((_RPT, H), _F32),
            pltpu.VMEM((16,), _F32),
            pltpu.VMEM_SHARED((_ROWS, H), _F32),
            pltpu.VMEM_SHARED((_ROWS, H), _F32),
        ] + [pltpu.SemaphoreType.DMA] * (2 * _NBUF),
        compiler_params=sc_params,
        interpret=interpret,
    )

    prep = pl.pallas_call(
        _prep_body,
        out_shape=jax.ShapeDtypeStruct((_ROWS, H), _F32),
        interpret=interpret,
    )
    final = pl.pallas_call(
        _final_body,
        out_shape=jax.ShapeDtypeStruct((N, C), _F32),
        interpret=interpret,
    )
    return agg1_k, agg2_k, prep, final


_FNS = None


def _run(fns, x, edge_index, W1, b1, W2, b2):
    agg1_k, agg2_k, prep, final = fns
    src = edge_index[0]
    dst = edge_index[1]
    npad = _EPAD - E
    src2d = jnp.concatenate(
        [src, jnp.zeros((npad,), jnp.int32)]).reshape(_NROW, _CHUNK)
    dst2d = jnp.concatenate(
        [dst, jnp.full((npad,), N, jnp.int32)]).reshape(_NROW, _CHUNK)
    w2p = jnp.zeros((H, H), _F32).at[:, :C].set(W2)
    b2p = jnp.zeros((H,), _F32).at[:C].set(b2)

    h1p = prep(x, W1)
    agg1p, dinv = agg1_k(h1p, src2d, dst2d)
    agg2p, w = agg2_k(h1p, src2d, dst2d, dinv, agg1p, b1)
    return final(agg2p[0, :N], agg2p[1, :N], w[:N], w2p, b2p)


def kernel(x, edge_index, W1, b1, W2, b2):
    global _FNS
    if _FNS is None:
        _FNS = _build()
    return _run(_FNS, x, edge_index, W1, b1, W2, b2)


# speedup vs baseline: 1.0088x; 1.0088x over previous
import functools

import jax
import jax.numpy as jnp
from jax import lax
from jax.experimental import pallas as pl
from jax.experimental.pallas import tpu as pltpu
from jax.experimental.pallas import tpu_sc as plsc

N = 10000
D = 128
H = 16
C = 7
E = 320000

_NC, _NS = 2, 16
_NT = _NC * _NS
_CHUNK = 128
_CPT = 80
_EPAD = _NT * _CPT * _CHUNK
_NROW = _EPAD // _CHUNK
_ROWS = 10240
_RPT = _ROWS // _NS
_DEG = _ROWS
_NBUF = 8
_PRE = 4
_HCPT = _NROW // _NS

_F32 = jnp.float32


def _splat(ref, i):
    return plsc.load_gather(ref, [jnp.full((16,), i, jnp.int32)])


def _newton_rsqrt(d):
    i = plsc.bitcast(d, jnp.int32)
    i = 0x5F3759DF - lax.shift_right_arithmetic(i, 1)
    y = plsc.bitcast(i, _F32)
    for _ in range(3):
        y = y * (1.5 - 0.5 * d * y * y)
    return y


def _load_edge_idx(src2d, dst2d, srcv, dstv, t):
    pltpu.sync_copy(src2d.at[pl.ds(t * _CPT, _CPT)], srcv)
    pltpu.sync_copy(dst2d.at[pl.ds(t * _CPT, _CPT)], dstv)


def _zero_spacc_slice(rows, spacc, s):
    zeros16 = jnp.zeros((16,), _F32)

    def zero_row(i, carry):
        for k in range(8):
            rows[0, i * 8 + k] = zeros16
        return carry

    lax.fori_loop(0, _CHUNK // 8, zero_row, 0)
    for q in range(_RPT // _CHUNK):
        pltpu.sync_copy(rows.at[0],
                        spacc.at[pl.ds(s * _RPT + q * _CHUNK, _CHUNK)])


def _ring(gtab, spacc, srcv, dstv, rows, gsems, ssems):
    gcp = [None] * _NBUF
    scp = [None] * _NBUF
    for b in range(_PRE):
        gcp[b] = pltpu.async_copy(gtab.at[srcv.at[b]], rows.at[b], gsems[b])
    for j in range(_CPT):
        b = j % _NBUF
        gcp[b].wait()
        scp[b] = pltpu.async_copy(
            rows.at[b], spacc.at[dstv.at[j]], ssems[b], add=True)
        f = j + _PRE
        if f < _CPT:
            bf = f % _NBUF
            if f >= _NBUF:
                scp[bf].wait()
            gcp[bf] = pltpu.async_copy(gtab.at[srcv.at[f]], rows.at[bf],
                                       gsems[bf])
    for j in range(max(_CPT - _NBUF, 0), _CPT):
        scp[j % _NBUF].wait()


def _agg1_body(h1p, src2d, dst2d, aggp, dinv_out,
               srcv, dstv, dsthist, hist, rows, degv, dinvv, hbuf, ubuf,
               dsem, gtab, spacc, spdeg, *sems):
    gsems = sems[:_NBUF]
    ssems = sems[_NBUF:]
    c = lax.axis_index("c")
    s = lax.axis_index("s")
    t = c * _NS + s
    _load_edge_idx(src2d, dst2d, srcv, dstv, t)
    zeros16 = jnp.zeros((16,), _F32)

    def hzero(i, carry):
        for k in range(8):
            hist[pl.ds((i * 8 + k) * 16, 16)] = zeros16
        return carry

    lax.fori_loop(0, _DEG // 128, hzero, 0)
    ones16 = jnp.ones((16,), _F32)

    def hchunk(j, carry):
        for k in range(_CHUNK // 16):
            idx = dsthist[j, pl.ds(k * 16, 16)]
            plsc.addupdate_scatter(hist, [idx], ones16)
        return carry

    hq = _HCPT // 4
    for q in range(4):
        pltpu.sync_copy(dst2d.at[pl.ds(s * _HCPT + q * hq, hq)], dsthist)
        lax.fori_loop(0, hq, hchunk, 0)
    pltpu.sync_copy(hist, spdeg.at[s])
    pltpu.sync_copy(h1p.at[pl.ds(s * _RPT, _RPT)], hbuf)
    _zero_spacc_slice(rows, spacc, s)
    plsc.subcore_barrier()
    for half in range(2):
        dcp = []
        for k in range(8):
            dcp.append(pltpu.async_copy(
                spdeg.at[half * 8 + k].at[pl.ds(s * _RPT, _RPT)],
                degv.at[k], dsem))
        for cp in dcp:
            cp.wait()
        for v in range(_RPT // 16):
            sl = pl.ds(v * 16, 16)
            acc = jnp.ones((16,), _F32) if half == 0 else dinvv[sl]
            for k in range(8):
                acc = acc + degv[k, sl]
            dinvv[sl] = acc if half == 0 else _newton_rsqrt(acc)
    pltpu.sync_copy(dinvv, dinv_out.at[pl.ds(s * _RPT, _RPT)])

    def scale_row(i4, carry):
        for r in range(4):
            i = i4 * 4 + r
            dv = _splat(dinvv, i)
            ubuf[i] = hbuf[i] * dv
        return carry

    lax.fori_loop(0, _RPT // 4, scale_row, 0)
    pltpu.sync_copy(ubuf, gtab.at[pl.ds(s * _RPT, _RPT)])
    plsc.subcore_barrier()
    _ring(gtab, spacc, srcv, dstv, rows, gsems, ssems)
    plsc.subcore_barrier()
    pltpu.sync_copy(spacc.at[pl.ds(s * _RPT, _RPT)],
                    aggp.at[c].at[pl.ds(s * _RPT, _RPT)])


def _agg2_body(h1p, src2d, dst2d, dinv_hbm, agg1p, b1, aggp, w_out,
               srcv, dstv, rows, dinvv, hbuf, p0buf, p1buf, ubuf, wbuf, bbuf,
               gtab, spacc, *sems):
    gsems = sems[:_NBUF]
    ssems = sems[_NBUF:]
    c = lax.axis_index("c")
    s = lax.axis_index("s")
    t = c * _NS + s
    _load_edge_idx(src2d, dst2d, srcv, dstv, t)
    pltpu.sync_copy(dinv_hbm.at[pl.ds(s * _RPT, _RPT)], dinvv)
    pltpu.sync_copy(h1p.at[pl.ds(s * _RPT, _RPT)], hbuf)
    pltpu.sync_copy(agg1p.at[0].at[pl.ds(s * _RPT, _RPT)], p0buf)
    pltpu.sync_copy(agg1p.at[1].at[pl.ds(s * _RPT, _RPT)], p1buf)
    pltpu.sync_copy(b1, bbuf)
    _zero_spacc_slice(rows, spacc, s)
    b1v = bbuf[...]

    def urow(i4, carry):
        for r in range(4):
            i = i4 * 4 + r
            dv = _splat(dinvv, i)
            g1r = hbuf[i] * dv
            srow = p0buf[i] + p1buf[i] + g1r
            z = jnp.maximum(srow * dv + b1v, 0.0)
            u = z * dv
            ubuf[i] = u
            wbuf[i] = u * dv
        return carry

    lax.fori_loop(0, _RPT // 4, urow, 0)
    pltpu.sync_copy(ubuf, gtab.at[pl.ds(s * _RPT, _RPT)])
    pltpu.sync_copy(wbuf, w_out.at[pl.ds(s * _RPT, _RPT)])
    plsc.subcore_barrier()
    _ring(gtab, spacc, srcv, dstv, rows, gsems, ssems)
    plsc.subcore_barrier()
    pltpu.sync_copy(spacc.at[pl.ds(s * _RPT, _RPT)], ubuf)

    def srow_(i4, carry):
        for r in range(4):
            i = i4 * 4 + r
            ubuf[i] = ubuf[i] * _splat(dinvv, i)
        return carry

    lax.fori_loop(0, _RPT // 4, srow_, 0)
    pltpu.sync_copy(ubuf, aggp.at[c].at[pl.ds(s * _RPT, _RPT)])


def _prep_body(x_ref, w1_ref, h1_ref):
    h1 = jnp.dot(x_ref[...], w1_ref[...], preferred_element_type=_F32)
    h1_ref[:N, :] = h1
    h1_ref[N:, :] = jnp.zeros((_ROWS - N, H), _F32)


def _final_body(p0_ref, p1_ref, w_ref, w2_ref, b2_ref, out_ref):
    t = p0_ref[...] + p1_ref[...] + w_ref[...]
    a2 = (jnp.dot(t, w2_ref[...], preferred_element_type=_F32)
          + b2_ref[...][None, :])
    col = lax.broadcasted_iota(jnp.int32, (N, H), 1)
    valid = col < C
    am = jnp.where(valid, a2, -1e30)
    m = jnp.max(am, axis=1, keepdims=True)
    e = jnp.where(valid, jnp.exp(am - m), 0.0)
    ssum = jnp.sum(e, axis=1, keepdims=True)
    ls = a2 - m - jnp.log(ssum)
    out_ref[...] = ls[:, :C]


def _build(interpret: bool = False):
    mesh = plsc.VectorSubcoreMesh(
        core_axis_name="c", subcore_axis_name="s",
        num_cores=_NC, num_subcores=_NS)
    sc_params = pltpu.CompilerParams(
        use_tc_tiling_on_sc=False, needs_layout_passes=False)

    agg1_k = pl.kernel(
        _agg1_body,
        out_type=[jax.ShapeDtypeStruct((_NC, _ROWS, H), _F32),
                  jax.ShapeDtypeStruct((_ROWS,), _F32)],
        mesh=mesh,
        scratch_types=[
            pltpu.VMEM((_CPT, _CHUNK), jnp.int32),
            pltpu.VMEM((_CPT, _CHUNK), jnp.int32),
            pltpu.VMEM((_HCPT // 4, _CHUNK), jnp.int32),
            pltpu.VMEM((_DEG,), _F32),
            pltpu.VMEM((_NBUF, _CHUNK, H), _F32),
            pltpu.VMEM((8, _RPT), _F32),
            pltpu.VMEM((_RPT,), _F32),
            pltpu.VMEM((_RPT, H), _F32),
            pltpu.VMEM((_RPT, H), _F32),
            pltpu.SemaphoreType.DMA,
            pltpu.VMEM_SHARED((_ROWS, H), _F32),
            pltpu.VMEM_SHARED((_ROWS, H), _F32),
            pltpu.VMEM_SHARED((_NS, _DEG), _F32),
        ] + [pltpu.SemaphoreType.DMA] * (2 * _NBUF),
        compiler_params=sc_params,
        interpret=interpret,
    )

    agg2_k = pl.kernel(
        _agg2_body,
        out_type=[jax.ShapeDtypeStruct((_NC, _ROWS, H), _F32),
                  jax.ShapeDtypeStruct((_ROWS, H), _F32)],
        mesh=mesh,
        scratch_types=[
            pltpu.VMEM((_CPT, _CHUNK), jnp.int32),
            pltpu.VMEM((_CPT, _CHUNK), jnp.int32),
            pltpu.VMEM((_NBUF, _CHUNK, H), _F32),
            pltpu.VMEM((_RPT,), _F32),
            pltpu.VMEM((_RPT, H), _F32),
            pltpu.VMEM((_RPT, H), _F32),
            pltpu.VMEM((_RPT, H), _F32),
            pltpu.VMEM((_RPT, H), _F32),
            pltpu.VMEM((_RPT, H), _F32),
            pltpu.VMEM((16,), _F32),
            pltpu.VMEM_SHARED((_ROWS, H), _F32),
            pltpu.VMEM_SHARED((_ROWS, H), _F32),
        ] + [pltpu.SemaphoreType.DMA] * (2 * _NBUF),
        compiler_params=sc_params,
        interpret=interpret,
    )

    prep = pl.pallas_call(
        _prep_body,
        out_shape=jax.ShapeDtypeStruct((_ROWS, H), _F32),
        interpret=interpret,
    )
    final = pl.pallas_call(
        _final_body,
        out_shape=jax.ShapeDtypeStruct((N, C), _F32),
        interpret=interpret,
    )
    return agg1_k, agg2_k, prep, final


_FNS = None


def _run(fns, x, edge_index, W1, b1, W2, b2):
    agg1_k, agg2_k, prep, final = fns
    src = edge_index[0]
    dst = edge_index[1]
    npad = _EPAD - E
    src2d = jnp.concatenate(
        [src, jnp.zeros((npad,), jnp.int32)]).reshape(_NROW, _CHUNK)
    dst2d = jnp.concatenate(
        [dst, jnp.full((npad,), N, jnp.int32)]).reshape(_NROW, _CHUNK)
    w2p = jnp.zeros((H, H), _F32).at[:, :C].set(W2)
    b2p = jnp.zeros((H,), _F32).at[:C].set(b2)

    h1p = prep(x, W1)
    agg1p, dinv = agg1_k(h1p, src2d, dst2d)
    agg2p, w = agg2_k(h1p, src2d, dst2d, dinv, agg1p, b1)
    return final(agg2p[0, :N], agg2p[1, :N], w[:N], w2p, b2p)


def kernel(x, edge_index, W1, b1, W2, b2):
    global _FNS
    if _FNS is None:
        _FNS = _build()
    return _run(_FNS, x, edge_index, W1, b1, W2, b2)
